# baseline (speedup 1.0000x reference)
import functools

import jax
import jax.numpy as jnp
from jax import lax
from jax.experimental import pallas as pl
from jax.experimental.pallas import tpu as pltpu
from jax.experimental.pallas import tpu_sc as plsc

EMBED_DIM = 64
LANES = 16
NUM_CORES = 2
NUM_SUBCORES = 16
NUM_WORKERS = NUM_CORES * NUM_SUBCORES


@functools.partial(jax.jit, static_argnames=("hist", "batch"))
def _sc_gather(xT, table, *, hist, batch):
    mesh = plsc.VectorSubcoreMesh(core_axis_name="c", subcore_axis_name="s")
    bw = batch // NUM_WORKERS
    nc = bw // 128
    n_units = hist * nc

    @functools.partial(
        pl.kernel,
        mesh=mesh,
        compiler_params=pltpu.CompilerParams(
            use_tc_tiling_on_sc=False, needs_layout_passes=False),
        out_type=jax.ShapeDtypeStruct((hist, EMBED_DIM, batch), jnp.float32),
        scratch_types=[
            pltpu.VMEM((hist, bw), jnp.int32),
            pltpu.VMEM((2, 128, EMBED_DIM), jnp.float32),
            pltpu.VMEM((2, EMBED_DIM, 128), jnp.float32),
        ] + [pltpu.SemaphoreType.DMA] * 4,
    )
    def k(x_hbm, tab_hbm, out_hbm, xv, g_v, t_v, gsem0, gsem1, osem0, osem1):
        wid = lax.axis_index("s") * NUM_CORES + lax.axis_index("c")
        b_base = wid * bw
        gsems = (gsem0, gsem1)
        osems = (osem0, osem1)
        iota = lax.broadcasted_iota(jnp.int32, (LANES,), 0)
        rowv = [iota + LANES * lq for lq in range(128 // LANES)]

        pltpu.sync_copy(x_hbm.at[:, pl.ds(b_base, bw)], xv)

        def start_gather(u, b):
            h = u // nc
            c = lax.rem(u, nc)
            pltpu.async_copy(
                tab_hbm.at[xv.at[h, pl.ds(c * 128, 128)]], g_v.at[b],
                gsems[b])

        def wait_gather(b):
            pltpu.make_async_copy(
                tab_hbm.at[pl.ds(0, 128)], g_v.at[b], gsems[b]).wait()

        def start_out(u, b):
            h = u // nc
            c = lax.rem(u, nc)
            pltpu.async_copy(
                t_v.at[b],
                out_hbm.at[h, :, pl.ds(b_base + c * 128, 128)], osems[b])

        def wait_out(b):
            pltpu.make_async_copy(
                t_v.at[b], out_hbm.at[0, :, pl.ds(0, 128)], osems[b]).wait()

        def transpose(b):
            @plsc.parallel_loop(0, 128, unroll=8)
            def body(l):
                colv = jnp.full((LANES,), 0, jnp.int32) + l
                for eq in range(EMBED_DIM // LANES):
                    vec = g_v[b, l, pl.ds(LANES * eq, LANES)]
                    plsc.store_scatter(t_v.at[b], [rowv[eq], colv], vec)

        start_gather(0, 0)
        start_gather(1, 1)

        def outer(uu, carry):
            for b in range(2):
                u = uu * 2 + b
                wait_gather(b)

                @pl.when(u >= 2)
                def _():
                    wait_out(b)

                transpose(b)
                start_out(u, b)

                @pl.when(u + 2 < n_units)
                def _():
                    start_gather(u + 2, b)
            return carry

        lax.fori_loop(0, n_units // 2, outer, 0)
        wait_out(0)
        wait_out(1)

    return k(xT, table)


def kernel(x, table):
    batch, hist = x.shape
    xT = jnp.transpose(x.astype(jnp.int32))
    outT = _sc_gather(xT, table, hist=hist, batch=batch)
    return jnp.transpose(outT, (2, 0, 1))


# speedup vs baseline: 1.1998x; 1.1998x over previous
import functools

import jax
import jax.numpy as jnp
from jax import lax
from jax.experimental import pallas as pl
from jax.experimental.pallas import tpu as pltpu
from jax.experimental.pallas import tpu_sc as plsc

EMBED_DIM = 64
NUM_CORES = 2
NUM_SUBCORES = 16
NUM_WORKERS = NUM_CORES * NUM_SUBCORES
NBUF = 8


@functools.partial(jax.jit, static_argnames=("rows_per_w", "hist"))
def _sc_gather(x, table, *, rows_per_w, hist):
    batch = NUM_WORKERS * rows_per_w
    mesh = plsc.VectorSubcoreMesh(core_axis_name="c", subcore_axis_name="s")

    @functools.partial(
        pl.kernel,
        mesh=mesh,
        compiler_params=pltpu.CompilerParams(use_tc_tiling_on_sc=False),
        out_type=jax.ShapeDtypeStruct((batch, hist, EMBED_DIM), jnp.float32),
        scratch_types=[
            pltpu.VMEM((rows_per_w, hist), jnp.int32),
            pltpu.VMEM((NBUF, hist, EMBED_DIM), jnp.float32),
        ] + [pltpu.SemaphoreType.DMA] * NBUF,
    )
    def k(x_hbm, table_hbm, out_hbm, idx_v, rows_v, *sems):
        wid = lax.axis_index("s") * NUM_CORES + lax.axis_index("c")
        base = wid * rows_per_w
        pltpu.sync_copy(x_hbm.at[pl.ds(base, rows_per_w)], idx_v)

        def start(i, b):
            pltpu.async_copy(table_hbm.at[idx_v.at[i]], rows_v.at[b], sems[b])

        def drain(i, b):
            pltpu.make_async_copy(
                table_hbm.at[pl.ds(0, hist)], rows_v.at[b], sems[b]
            ).wait()
            pltpu.sync_copy(rows_v.at[b], out_hbm.at[base + i])

        for b in range(NBUF):
            start(b, b)

        def outer(j, carry):
            i0 = j * NBUF
            for b in range(NBUF):
                drain(i0 + b, b)
                start(i0 + b + NBUF, b)
            return carry

        lax.fori_loop(0, rows_per_w // NBUF - 1, outer, 0)
        for b in range(NBUF):
            drain(rows_per_w - NBUF + b, b)

    return k(x, table)


def kernel(x, table):
    batch, hist = x.shape
    return _sc_gather(
        x.astype(jnp.int32), table,
        rows_per_w=batch // NUM_WORKERS, hist=hist,
    )
